# baseline (speedup 1.0000x reference)
import jax
import jax.numpy as jnp
from jax import lax
from jax.experimental import pallas as pl
from jax.experimental.pallas import tpu as pltpu
from jax.experimental.pallas import tpu_sc as plsc

_BINS = 100
_NEDGES = _BINS - 1
_LO, _HI = -3.0, 3.0
_MEAN, _STD = 0.0, 1.0
_INV_STEP = float(_NEDGES - 1) / (_HI - _LO)

_BATCH = 16384
_NFLAT = 2 * _BATCH
_NW = 32
_PER_W = _NFLAT // _NW
_LANES = 16
_NBLK = _PER_W // _LANES
_TROWS = 2 * _BINS
_PITCH = 256
_CBLKS = 16
_CROWS = _CBLKS * _LANES // 2
_CWORDS = _CROWS * _PITCH
_NCHUNK = _NBLK // _CBLKS

_mesh = plsc.VectorSubcoreMesh(
    core_axis_name="c", subcore_axis_name="s", num_cores=2, num_subcores=16)


def _sc_body(crs_hbm, table_hbm, ep_hbm, out_hbm, crs_v, ep_v, idx_v,
             table_v, stage_v, sem_a, sem_b):
    wid = lax.axis_index("s") * 2 + lax.axis_index("c")
    base = wid * _PER_W
    pltpu.sync_copy(crs_hbm.at[pl.ds(base, _PER_W)], crs_v)
    pltpu.sync_copy(ep_hbm, ep_v)
    pltpu.sync_copy(table_hbm, table_v)

    iota = lax.iota(jnp.int32, _LANES)
    src_lane = iota ^ 1
    parity_add = jnp.where((iota & 1) == 0, jnp.int32(0), jnp.int32(_BINS))

    for j in range(_NBLK):
        x = plsc.load_gather(crs_v, [jnp.int32(j * _LANES) + src_lane])
        x = (x - _MEAN) / _STD
        c = jnp.clip((x - _LO) * _INV_STEP, 0.0, float(_BINS - 1))
        c = c.astype(jnp.int32) + 1
        c = jnp.clip(c, 0, _BINS - 1)
        e0 = plsc.load_gather(ep_v, [c])
        e1 = plsc.load_gather(ep_v, [c + 1])
        one = jnp.int32(1)
        zero = jnp.int32(0)
        idx = (c - 1 + jnp.where(x >= e0, one, zero)
               + jnp.where(x >= e1, one, zero))
        idx_v[pl.ds(j * _LANES, _LANES)] = (idx + parity_add) * _BINS

    _G = 10
    dst_lane0 = (iota >> 1) * _PITCH + (iota & 1) * _BINS

    def make_block(buf):
        def block(b, carry):
            src0 = idx_v[pl.ds(b * _LANES, _LANES)]
            dst0 = ((b % _CBLKS) * ((_LANES // 2) * _PITCH)
                    + buf * _CWORDS) + dst_lane0
            for c0 in range(0, _BINS, _G):
                vals = [plsc.load_gather(table_v, [src0 + (c0 + g)])
                        for g in range(_G)]
                for g in range(_G):
                    plsc.store_scatter(stage_v, [dst0 + (c0 + g)], vals[g])
            return carry
        return block

    inflight = [None, None]
    for ch in range(_NCHUNK):
        buf = ch % 2
        if inflight[buf] is not None:
            inflight[buf].wait()
        lax.fori_loop(ch * _CBLKS, (ch + 1) * _CBLKS, make_block(buf), 0,
                      unroll=False)
        inflight[buf] = pltpu.async_copy(
            stage_v.at[pl.ds(buf * _CWORDS, _CWORDS)],
            out_hbm.at[pl.ds((base // 2) * _PITCH + ch * _CWORDS, _CWORDS)],
            sem_a if buf == 0 else sem_b,
        )
    for cp in inflight:
        if cp is not None:
            cp.wait()


_sc_lookup = pl.kernel(
    _sc_body,
    mesh=_mesh,
    out_type=jax.ShapeDtypeStruct((_BATCH * _PITCH,), jnp.float32),
    scratch_types=[
        pltpu.VMEM((_PER_W,), jnp.float32),
        pltpu.VMEM((128,), jnp.float32),
        pltpu.VMEM((_PER_W,), jnp.int32),
        pltpu.VMEM((_TROWS * _BINS,), jnp.float32),
        pltpu.VMEM((2 * _CWORDS,), jnp.float32),
        pltpu.SemaphoreType.DMA,
        pltpu.SemaphoreType.DMA,
    ],
    compiler_params=pltpu.CompilerParams(needs_layout_passes=False),
)

_RB = 1024


def _tc_depad_body(in_ref, out_ref):
    blk = in_ref[...].reshape(_RB, _PITCH)
    out_ref[...] = blk[:, :2 * _BINS]


_tc_depad = pl.pallas_call(
    _tc_depad_body,
    grid=(_BATCH // _RB,),
    in_specs=[pl.BlockSpec((_RB * _PITCH,), lambda i: (i,))],
    out_specs=pl.BlockSpec((_RB, 2 * _BINS), lambda i: (i, 0)),
    out_shape=jax.ShapeDtypeStruct((_BATCH, 2 * _BINS), jnp.float32),
)


@jax.jit
def kernel(crs, lat_table, lon_table):
    table = jnp.concatenate([lat_table, lon_table], axis=0)
    edges = jnp.linspace(_LO, _HI, _NEDGES)
    ep = jnp.concatenate([
        jnp.array([-jnp.inf], jnp.float32),
        edges.astype(jnp.float32),
        jnp.full((128 - _NEDGES - 1,), jnp.inf, jnp.float32),
    ])
    padded = _sc_lookup(crs.reshape(_NFLAT), table.reshape(-1), ep)
    return _tc_depad(padded)


# speedup vs baseline: 1.5760x; 1.5760x over previous
import jax
import jax.numpy as jnp
from jax import lax
from jax.experimental import pallas as pl
from jax.experimental.pallas import tpu as pltpu
from jax.experimental.pallas import tpu_sc as plsc

_BINS = 100
_NEDGES = _BINS - 1
_LO, _HI = -3.0, 3.0
_MEAN, _STD = 0.0, 1.0
_INV_STEP = float(_NEDGES - 1) / (_HI - _LO)

_BATCH = 16384
_NFLAT = 2 * _BATCH
_NW = 32
_PER_W = _NFLAT // _NW
_LANES = 16
_NBLK = _PER_W // _LANES
_TROWS = 2 * _BINS
_CBLKS = 16
_CROWS = _CBLKS * _LANES // 2
_CWORDS = _TROWS * _CROWS
_NCHUNK = _NBLK // _CBLKS

_mesh = plsc.VectorSubcoreMesh(
    core_axis_name="c", subcore_axis_name="s", num_cores=2, num_subcores=16)


def _sc_body(crs_hbm, table_hbm, ep_hbm, out_hbm, crs_v, ep_v, idx_v,
             table_v, stage_v, sem_a, sem_b):
    wid = lax.axis_index("s") * 2 + lax.axis_index("c")
    base = wid * _PER_W
    pltpu.sync_copy(crs_hbm.at[pl.ds(base, _PER_W)], crs_v)
    pltpu.sync_copy(ep_hbm, ep_v)
    pltpu.sync_copy(table_hbm, table_v)

    iota = lax.iota(jnp.int32, _LANES)
    src_lane = iota ^ 1
    parity_add = jnp.where((iota & 1) == 0, jnp.int32(0), jnp.int32(_BINS))

    for j in range(_NBLK):
        x = plsc.load_gather(crs_v, [jnp.int32(j * _LANES) + src_lane])
        x = (x - _MEAN) / _STD
        c = jnp.clip((x - _LO) * _INV_STEP, 0.0, float(_BINS - 1))
        c = c.astype(jnp.int32) + 1
        c = jnp.clip(c, 0, _BINS - 1)
        e0 = plsc.load_gather(ep_v, [c])
        e1 = plsc.load_gather(ep_v, [c + 1])
        one = jnp.int32(1)
        zero = jnp.int32(0)
        idx = (c - 1 + jnp.where(x >= e0, one, zero)
               + jnp.where(x >= e1, one, zero))
        idx_v[pl.ds(j * _LANES, _LANES)] = (idx + parity_add) * _BINS

    _G = 10
    dbase = (iota & 1) * _BINS
    nbase0 = iota >> 1

    def make_block(buf):
        def block(b, carry):
            src0 = idx_v[pl.ds(b * _LANES, _LANES)]
            nvec = ((b % _CBLKS) * (_LANES // 2)) + nbase0
            for c0 in range(0, _BINS, _G):
                vals = [plsc.load_gather(table_v, [src0 + (c0 + g)])
                        for g in range(_G)]
                for g in range(_G):
                    plsc.store_scatter(
                        stage_v.at[buf], [dbase + (c0 + g), nvec], vals[g])
            return carry
        return block

    inflight = [None, None]
    for ch in range(_NCHUNK):
        buf = ch % 2
        if inflight[buf] is not None:
            inflight[buf].wait()
        lax.fori_loop(ch * _CBLKS, (ch + 1) * _CBLKS, make_block(buf), 0,
                      unroll=False)
        inflight[buf] = pltpu.async_copy(
            stage_v.at[buf],
            out_hbm.at[:, pl.ds(pl.multiple_of((base // 2) + ch * _CROWS, _CROWS), _CROWS)],
            sem_a if buf == 0 else sem_b,
        )
    for cp in inflight:
        if cp is not None:
            cp.wait()


_sc_lookup = pl.kernel(
    _sc_body,
    mesh=_mesh,
    out_type=jax.ShapeDtypeStruct((_TROWS, _BATCH), jnp.float32),
    scratch_types=[
        pltpu.VMEM((_PER_W,), jnp.float32),
        pltpu.VMEM((128,), jnp.float32),
        pltpu.VMEM((_PER_W,), jnp.int32),
        pltpu.VMEM((_TROWS * _BINS,), jnp.float32),
        pltpu.VMEM((2, _TROWS, _CROWS), jnp.float32),
        pltpu.SemaphoreType.DMA,
        pltpu.SemaphoreType.DMA,
    ],
    compiler_params=pltpu.CompilerParams(needs_layout_passes=False),
)

@jax.jit
def kernel(crs, lat_table, lon_table):
    table = jnp.concatenate([lat_table, lon_table], axis=0)
    edges = jnp.linspace(_LO, _HI, _NEDGES)
    ep = jnp.concatenate([
        jnp.array([-jnp.inf], jnp.float32),
        edges.astype(jnp.float32),
        jnp.full((128 - _NEDGES - 1,), jnp.inf, jnp.float32),
    ])
    out_t = _sc_lookup(crs.reshape(_NFLAT), table.reshape(-1), ep)
    return out_t.T
